# baseline (speedup 1.0000x reference)
import functools

import jax
import jax.numpy as jnp
from jax import lax
from jax.experimental import pallas as pl
from jax.experimental.pallas import tpu as pltpu
from jax.experimental.pallas import tpu_sc as plsc

B, L, D = 4096, 201, 128
V = 3

NC, NS = 2, 16
NW = NC * NS
BPW = B // NW
RPW = BPW * L
NT = RPW // 16

RB = 2048
NB = (B * L) // RB


def _sc_body(ids_hbm, len_hbm, ids2_hbm, clen_hbm,
             ids_v, len_v, len1_v, eofidx_v, eofmark_v, sem):
    wid = lax.axis_index("s") * NC + lax.axis_index("c")
    base_b = wid * BPW
    base_r = wid * RPW

    pltpu.sync_copy(ids_hbm.at[pl.ds(base_r, RPW)], ids_v)
    pltpu.sync_copy(len_hbm.at[pl.ds(base_b, BPW)], len_v)

    eof_row = jnp.full((16,), V, dtype=jnp.int32)
    for k in range(BPW // 16):
        lv = len_v[pl.ds(k * 16, 16)]
        jb = lax.iota(jnp.int32, 16) + (k * 16)
        eofidx_v[pl.ds(k * 16, 16)] = base_r + jb * L + lv
        eofmark_v[pl.ds(k * 16, 16)] = eof_row
        len1_v[pl.ds(k * 16, 16)] = lv + 1
    pltpu.sync_copy(len1_v, clen_hbm.at[pl.ds(base_b, BPW)])

    pltpu.sync_copy(ids_v, ids2_hbm.at[pl.ds(base_r, RPW)])
    pltpu.async_copy(eofmark_v, ids2_hbm.at[eofidx_v], sem).wait()


_sc_call = pl.kernel(
    _sc_body,
    out_type=(
        jax.ShapeDtypeStruct((B * L,), jnp.int32),
        jax.ShapeDtypeStruct((B,), jnp.int32),
    ),
    mesh=plsc.VectorSubcoreMesh(core_axis_name="c", subcore_axis_name="s"),
    scratch_types=[
        pltpu.VMEM((RPW,), jnp.int32),
        pltpu.VMEM((BPW,), jnp.int32),
        pltpu.VMEM((BPW,), jnp.int32),
        pltpu.VMEM((BPW,), jnp.int32),
        pltpu.VMEM((BPW,), jnp.int32),
        pltpu.SemaphoreType.DMA,
    ],
)


def _tc_body(ids_ref, tab_ref, out_ref):
    iv = ids_ref[0, 0, :]
    w = iv[:, None]
    t0 = tab_ref[0, :][None, :]
    t1 = tab_ref[1, :][None, :]
    t2 = tab_ref[2, :][None, :]
    t3 = tab_ref[3, :][None, :]
    out_ref[...] = jnp.where(
        w < 2,
        jnp.where(w == 0, t0, t1),
        jnp.where(w == 2, t2, t3),
    )


_tc_call = pl.pallas_call(
    _tc_body,
    grid=(NB,),
    in_specs=[
        pl.BlockSpec((1, 1, RB), lambda i: (i, 0, 0)),
        pl.BlockSpec((V + 1, D), lambda i: (0, 0)),
    ],
    out_specs=pl.BlockSpec((RB, D), lambda i: (i, 0)),
    out_shape=jax.ShapeDtypeStruct((B * L, D), jnp.float32),
)


@jax.jit
def kernel(word_ids, lengths, table, eof_embedding):
    ids_flat = word_ids.reshape(B * L)
    table4 = jnp.concatenate([table, eof_embedding], axis=0)
    ids2, char_len = _sc_call(ids_flat, lengths)
    rep_flat = _tc_call(ids2.reshape(NB, 1, RB), table4)
    return rep_flat.reshape(B, L, D), char_len


# speedup vs baseline: 1.7156x; 1.7156x over previous
import functools

import jax
import jax.numpy as jnp
from jax import lax
from jax.experimental import pallas as pl
from jax.experimental.pallas import tpu as pltpu
from jax.experimental.pallas import tpu_sc as plsc

B, L, D = 4096, 201, 128
V = 3

NC, NS = 2, 16
NW = NC * NS
BPW = B // NW
RPW = BPW * L
NT = RPW // 16

BB = 16
NTC = B // BB


def _sc_body(ids_hbm, len_hbm, ids2_hbm, clen_hbm,
             ids_v, len_v, len1_v, eofidx_v, eofmark_v, sem):
    wid = lax.axis_index("s") * NC + lax.axis_index("c")
    base_b = wid * BPW
    base_r = wid * RPW

    pltpu.sync_copy(ids_hbm.at[pl.ds(base_r, RPW)], ids_v)
    pltpu.sync_copy(len_hbm.at[pl.ds(base_b, BPW)], len_v)

    eof_row = jnp.full((16,), V, dtype=jnp.int32)
    for k in range(BPW // 16):
        lv = len_v[pl.ds(k * 16, 16)]
        jb = lax.iota(jnp.int32, 16) + (k * 16)
        eofidx_v[pl.ds(k * 16, 16)] = base_r + jb * L + lv
        eofmark_v[pl.ds(k * 16, 16)] = eof_row
        len1_v[pl.ds(k * 16, 16)] = lv + 1
    pltpu.sync_copy(len1_v, clen_hbm.at[pl.ds(base_b, BPW)])

    pltpu.sync_copy(ids_v, ids2_hbm.at[pl.ds(base_r, RPW)])
    pltpu.async_copy(eofmark_v, ids2_hbm.at[eofidx_v], sem).wait()


_sc_call = pl.kernel(
    _sc_body,
    out_type=(
        jax.ShapeDtypeStruct((B * L,), jnp.int32),
        jax.ShapeDtypeStruct((B,), jnp.int32),
    ),
    mesh=plsc.VectorSubcoreMesh(core_axis_name="c", subcore_axis_name="s"),
    scratch_types=[
        pltpu.VMEM((RPW,), jnp.int32),
        pltpu.VMEM((BPW,), jnp.int32),
        pltpu.VMEM((BPW,), jnp.int32),
        pltpu.VMEM((BPW,), jnp.int32),
        pltpu.VMEM((BPW,), jnp.int32),
        pltpu.SemaphoreType.DMA,
    ],
)


def _tc_body(ids_ref, tab_ref, out_ref):
    w = ids_ref[...][:, :, None]
    t0 = tab_ref[0, :][None, None, :]
    t1 = tab_ref[1, :][None, None, :]
    t2 = tab_ref[2, :][None, None, :]
    t3 = tab_ref[3, :][None, None, :]
    out_ref[...] = jnp.where(
        w < 2,
        jnp.where(w == 0, t0, t1),
        jnp.where(w == 2, t2, t3),
    )


_tc_call = pl.pallas_call(
    _tc_body,
    grid=(NTC,),
    in_specs=[
        pl.BlockSpec((BB, L), lambda i: (i, 0)),
        pl.BlockSpec((V + 1, D), lambda i: (0, 0)),
    ],
    out_specs=pl.BlockSpec((BB, L, D), lambda i: (i, 0, 0)),
    out_shape=jax.ShapeDtypeStruct((B, L, D), jnp.float32),
)


@jax.jit
def kernel(word_ids, lengths, table, eof_embedding):
    ids_flat = word_ids.reshape(B * L)
    table4 = jnp.concatenate([table, eof_embedding], axis=0)
    ids2, char_len = _sc_call(ids_flat, lengths)
    rep = _tc_call(ids2.reshape(B, L), table4)
    return rep, char_len


# speedup vs baseline: 2.0551x; 1.1979x over previous
import functools

import jax
import jax.numpy as jnp
from jax import lax
from jax.experimental import pallas as pl
from jax.experimental.pallas import tpu as pltpu
from jax.experimental.pallas import tpu_sc as plsc

B, L, D = 4096, 201, 128
V = 3

NC, NS = 2, 16
NW = NC * NS
BPW = B // NW
RPW = BPW * L
NT = RPW // 16

BB = 64
NTC = B // BB


def _sc_body(ids_hbm, len_hbm, ids2_hbm, clen_hbm,
             ids_v, len_v, len1_v, eofidx_v, eofmark_v, sem):
    wid = lax.axis_index("s") * NC + lax.axis_index("c")
    base_b = wid * BPW
    base_r = wid * RPW

    pltpu.sync_copy(ids_hbm.at[pl.ds(base_r, RPW)], ids_v)
    pltpu.sync_copy(len_hbm.at[pl.ds(base_b, BPW)], len_v)

    eof_row = jnp.full((16,), V, dtype=jnp.int32)
    for k in range(BPW // 16):
        lv = len_v[pl.ds(k * 16, 16)]
        jb = lax.iota(jnp.int32, 16) + (k * 16)
        eofidx_v[pl.ds(k * 16, 16)] = base_r + jb * L + lv
        eofmark_v[pl.ds(k * 16, 16)] = eof_row
        len1_v[pl.ds(k * 16, 16)] = lv + 1
    pltpu.sync_copy(len1_v, clen_hbm.at[pl.ds(base_b, BPW)])

    pltpu.sync_copy(ids_v, ids2_hbm.at[pl.ds(base_r, RPW)])
    pltpu.async_copy(eofmark_v, ids2_hbm.at[eofidx_v], sem).wait()


_sc_call = pl.kernel(
    _sc_body,
    out_type=(
        jax.ShapeDtypeStruct((B * L,), jnp.int32),
        jax.ShapeDtypeStruct((B,), jnp.int32),
    ),
    mesh=plsc.VectorSubcoreMesh(core_axis_name="c", subcore_axis_name="s"),
    scratch_types=[
        pltpu.VMEM((RPW,), jnp.int32),
        pltpu.VMEM((BPW,), jnp.int32),
        pltpu.VMEM((BPW,), jnp.int32),
        pltpu.VMEM((BPW,), jnp.int32),
        pltpu.VMEM((BPW,), jnp.int32),
        pltpu.SemaphoreType.DMA,
    ],
)


def _tc_body(ids_ref, tab_ref, out_ref):
    w = ids_ref[...][:, :, None]
    t0 = tab_ref[0, :][None, None, :]
    t1 = tab_ref[1, :][None, None, :]
    t2 = tab_ref[2, :][None, None, :]
    t3 = tab_ref[3, :][None, None, :]
    out_ref[...] = jnp.where(
        w < 2,
        jnp.where(w == 0, t0, t1),
        jnp.where(w == 2, t2, t3),
    )


_tc_call = pl.pallas_call(
    _tc_body,
    grid=(NTC,),
    in_specs=[
        pl.BlockSpec((BB, L), lambda i: (i, 0)),
        pl.BlockSpec((V + 1, D), lambda i: (0, 0)),
    ],
    out_specs=pl.BlockSpec((BB, L, D), lambda i: (i, 0, 0)),
    out_shape=jax.ShapeDtypeStruct((B, L, D), jnp.float32),
)


@jax.jit
def kernel(word_ids, lengths, table, eof_embedding):
    ids_flat = word_ids.reshape(B * L)
    table4 = jnp.concatenate([table, eof_embedding], axis=0)
    ids2, char_len = _sc_call(ids_flat, lengths)
    rep = _tc_call(ids2.reshape(B, L), table4)
    return rep, char_len


# speedup vs baseline: 2.1007x; 1.0222x over previous
import functools

import jax
import jax.numpy as jnp
from jax import lax
from jax.experimental import pallas as pl
from jax.experimental.pallas import tpu as pltpu
from jax.experimental.pallas import tpu_sc as plsc

B, L, D = 4096, 201, 128
V = 3

NC, NS = 2, 16
NW = NC * NS
BPW = B // NW
RPW = BPW * L
NT = RPW // 16

BB = 128
NTC = B // BB


def _sc_body(ids_hbm, len_hbm, ids2_hbm, clen_hbm,
             ids_v, len_v, len1_v, eofidx_v, eofmark_v, sem):
    wid = lax.axis_index("s") * NC + lax.axis_index("c")
    base_b = wid * BPW
    base_r = wid * RPW

    pltpu.sync_copy(ids_hbm.at[pl.ds(base_r, RPW)], ids_v)
    pltpu.sync_copy(len_hbm.at[pl.ds(base_b, BPW)], len_v)

    eof_row = jnp.full((16,), V, dtype=jnp.int32)
    for k in range(BPW // 16):
        lv = len_v[pl.ds(k * 16, 16)]
        jb = lax.iota(jnp.int32, 16) + (k * 16)
        eofidx_v[pl.ds(k * 16, 16)] = base_r + jb * L + lv
        eofmark_v[pl.ds(k * 16, 16)] = eof_row
        len1_v[pl.ds(k * 16, 16)] = lv + 1
    pltpu.sync_copy(len1_v, clen_hbm.at[pl.ds(base_b, BPW)])

    pltpu.sync_copy(ids_v, ids2_hbm.at[pl.ds(base_r, RPW)])
    pltpu.async_copy(eofmark_v, ids2_hbm.at[eofidx_v], sem).wait()


_sc_call = pl.kernel(
    _sc_body,
    out_type=(
        jax.ShapeDtypeStruct((B * L,), jnp.int32),
        jax.ShapeDtypeStruct((B,), jnp.int32),
    ),
    mesh=plsc.VectorSubcoreMesh(core_axis_name="c", subcore_axis_name="s"),
    scratch_types=[
        pltpu.VMEM((RPW,), jnp.int32),
        pltpu.VMEM((BPW,), jnp.int32),
        pltpu.VMEM((BPW,), jnp.int32),
        pltpu.VMEM((BPW,), jnp.int32),
        pltpu.VMEM((BPW,), jnp.int32),
        pltpu.SemaphoreType.DMA,
    ],
)


def _tc_body(ids_ref, tab_ref, out_ref):
    w = ids_ref[...][:, :, None]
    t0 = tab_ref[0, :][None, None, :]
    t1 = tab_ref[1, :][None, None, :]
    t2 = tab_ref[2, :][None, None, :]
    t3 = tab_ref[3, :][None, None, :]
    out_ref[...] = jnp.where(
        w < 2,
        jnp.where(w == 0, t0, t1),
        jnp.where(w == 2, t2, t3),
    )


_tc_call = pl.pallas_call(
    _tc_body,
    grid=(NTC,),
    in_specs=[
        pl.BlockSpec((BB, L), lambda i: (i, 0)),
        pl.BlockSpec((V + 1, D), lambda i: (0, 0)),
    ],
    out_specs=pl.BlockSpec((BB, L, D), lambda i: (i, 0, 0)),
    out_shape=jax.ShapeDtypeStruct((B, L, D), jnp.float32),
)


@jax.jit
def kernel(word_ids, lengths, table, eof_embedding):
    ids_flat = word_ids.reshape(B * L)
    table4 = jnp.concatenate([table, eof_embedding], axis=0)
    ids2, char_len = _sc_call(ids_flat, lengths)
    rep = _tc_call(ids2.reshape(B, L), table4)
    return rep, char_len
